# baseline (speedup 1.0000x reference)
import functools

import jax
import jax.numpy as jnp
from jax import lax
from jax.experimental import pallas as pl
from jax.experimental.pallas import tpu as pltpu
from jax.experimental.pallas import tpu_sc as plsc

_N = 10000
_E = 160000
_NUM_GENE = 120000
_NUM_DRUG = _E - _NUM_GENE

_NC, _NS, _L = 2, 16, 16
_EPT = _E // _NS
_BB = 80
_NBATCH = _EPT // _BB
_NPAD = 10240
_RPT = _NPAD // _NS
_ZR = 128
_CW = 64
_NCH = 512 // _CW


def _matmul12(h, W):
    N, K = h.shape
    RB = 1000

    def body(h_ref, w_ref, o_ref):
        o_ref[0] = jnp.dot(h_ref[...], w_ref[...],
                           preferred_element_type=jnp.float32)

    return pl.pallas_call(
        body,
        grid=(N // RB, 12),
        in_specs=[
            pl.BlockSpec((RB, K), lambda i, j: (i, 0)),
            pl.BlockSpec((K, 128), lambda i, j: (0, j)),
        ],
        out_specs=pl.BlockSpec((1, RB, 128), lambda i, j: (j, i, 0)),
        out_shape=jax.ShapeDtypeStruct((12, N, 128), jnp.float32),
    )(h, W)


def _sc_aggregate(tab, ei_r, w_r, do_deg):
    mesh = plsc.VectorSubcoreMesh(core_axis_name="c", subcore_axis_name="s")

    @functools.partial(
        pl.kernel,
        out_type=([jax.ShapeDtypeStruct((2, 4, _NPAD, 128), jnp.float32),
                   jax.ShapeDtypeStruct((2, _NPAD, _CW), jnp.float32)]
                  if do_deg else
                  jax.ShapeDtypeStruct((2, 4, _NPAD, 128), jnp.float32)),
        mesh=mesh,
        compiler_params=pltpu.CompilerParams(use_tc_tiling_on_sc=False),
        scratch_types=[
            pltpu.VMEM((_NBATCH, _BB), jnp.int32),
            pltpu.VMEM((_NBATCH, _BB), jnp.int32),
            pltpu.VMEM((_NBATCH, _BB), jnp.float32),
            pltpu.VMEM((4, _BB, _CW), jnp.float32),
            pltpu.VMEM((_ZR, _CW), jnp.float32),
            pltpu.VMEM((_ZR, _CW), jnp.float32),
            pltpu.VMEM((1, _BB), jnp.int32),
            pltpu.VMEM_SHARED((_NPAD, _CW), jnp.float32),
            [pltpu.SemaphoreType.DMA] * 4,
            [pltpu.SemaphoreType.DMA] * 4,
        ],
    )
    def agg(tab_h, ei_h, w_h, out_h, *rest):
        if do_deg:
            (deg_h, gidx_v, sidx_v, w_v, rows_v,
             z_v, cp_v, pad_v, acc_sh, gsems, ssems) = rest
        else:
            (gidx_v, sidx_v, w_v, rows_v,
             z_v, cp_v, pad_v, acc_sh, gsems, ssems) = rest
        c = lax.axis_index("c")
        s = lax.axis_index("s")
        pltpu.sync_copy(ei_h.at[c, s], gidx_v)
        pltpu.sync_copy(ei_h.at[1 - c, s], sidx_v)
        pltpu.sync_copy(w_h.at[c, s], w_v)
        zv = jnp.zeros((_L,), jnp.float32)
        for r in range(_ZR):
            for j in range(_CW // _L):
                z_v[r, pl.ds(j * _L, _L)] = zv
        padv = jnp.full((_L,), _NPAD - 8, jnp.int32)
        for j in range(_BB // _L):
            pad_v[0, pl.ds(j * _L, _L)] = padv

        def chunk_body(ch, carry):
            for r in range(_RPT // _ZR):
                pltpu.sync_copy(
                    z_v, acc_sh.at[pl.ds(s * _RPT + r * _ZR, _ZR)])
            plsc.subcore_barrier()

            def gather(bi, slot):
                pltpu.async_copy(
                    tab_h.at[c, ch].at[gidx_v.at[bi]], rows_v.at[slot],
                    gsems[slot])

            def wait_gather(slot):
                pltpu.make_async_copy(
                    tab_h.at[c, ch].at[gidx_v.at[0]], rows_v.at[slot],
                    gsems[slot]).wait()

            def scatter(bi, slot):
                pltpu.async_copy(rows_v.at[slot], acc_sh.at[sidx_v.at[bi]],
                                 ssems[slot], add=True)

            def wait_scatter(slot):
                pltpu.make_async_copy(rows_v.at[slot],
                                      acc_sh.at[sidx_v.at[0]],
                                      ssems[slot]).wait()

            def scale(bi, slot):
                def scale16(g, carry2):
                    w16 = w_v[bi, pl.ds(g * 16, 16)]
                    for b in range(16):
                        e = g * 16 + b
                        w_s = w16[b]
                        for j in range(_CW // _L):
                            sl = pl.ds(j * _L, _L)
                            rows_v[slot, e, sl] = rows_v[slot, e, sl] * w_s
                    return carry2
                lax.fori_loop(0, _BB // 16, scale16, 0)

            for k in (1, 2, 3):
                pltpu.async_copy(rows_v.at[k], acc_sh.at[pad_v.at[0]],
                                 ssems[k], add=True)
            for k in (0, 1, 2):
                gather(k, k)

            def group_body(p, carry2):
                for q in range(4):
                    bi = p * 4 + q
                    wait_gather(q)
                    scale(bi, q)
                    scatter(bi, q)
                    pk = (q + 3) % 4
                    wait_scatter(pk)

                    @pl.when(bi + 3 < _NBATCH)
                    def _():
                        gather(bi + 3, pk)
                return carry2

            lax.fori_loop(0, (_NBATCH - 1) // 4, group_body, 0)
            wait_gather(0)
            scale(_NBATCH - 1, 0)
            scatter(_NBATCH - 1, 0)
            for k in range(4):
                wait_scatter(k)

            plsc.subcore_barrier()
            ch128 = lax.div(ch, 2)
            hf = lax.rem(ch, 2)
            for r in range(_RPT // _ZR):
                row0 = s * _RPT + r * _ZR
                pltpu.sync_copy(acc_sh.at[pl.ds(row0, _ZR)], cp_v)
                pltpu.sync_copy(
                    cp_v,
                    out_h.at[c, ch128, pl.ds(row0, _ZR),
                             pl.ds(hf * _CW, _CW)])
            return carry

        lax.fori_loop(0, _NCH, chunk_body, 0)

        if not do_deg:
            return
        one = jnp.full((_L,), 1.0, jnp.float32)
        for j in range(_CW // _L):
            z_v[0, pl.ds(j * _L, _L)] = one

        def ones_fill(r, carry):
            for j in range(_CW // _L):
                rows_v[0, r, pl.ds(j * _L, _L)] = z_v[0, pl.ds(j * _L, _L)]
            return carry
        lax.fori_loop(0, _BB, ones_fill, 0)
        zv2 = jnp.zeros((_L,), jnp.float32)
        for j in range(_CW // _L):
            z_v[0, pl.ds(j * _L, _L)] = zv2
        for r in range(_RPT // _ZR):
            pltpu.sync_copy(z_v, acc_sh.at[pl.ds(s * _RPT + r * _ZR, _ZR)])
        plsc.subcore_barrier()

        def dscatter(bi, k):
            pltpu.async_copy(rows_v.at[0], acc_sh.at[sidx_v.at[bi]],
                             ssems[k], add=True)

        def dwait(k):
            pltpu.make_async_copy(rows_v.at[0], acc_sh.at[sidx_v.at[0]],
                                  ssems[k]).wait()

        def deg_group(p, carry):
            for q in range(4):
                dscatter(p * 4 + q, q)
            for q in range(4):
                dwait(q)
            return carry
        lax.fori_loop(0, (_NBATCH - 1) // 4, deg_group, 0)
        dscatter(_NBATCH - 1, 0)
        dwait(0)
        plsc.subcore_barrier()
        for r in range(_RPT // _ZR):
            row0 = s * _RPT + r * _ZR
            pltpu.sync_copy(acc_sh.at[pl.ds(row0, _ZR)], cp_v)
            pltpu.sync_copy(cp_v, deg_h.at[c].at[pl.ds(row0, _ZR)])

    return agg(tab, ei_r, w_r)


def _epilogue_c(s, proj12, inv_up, inv_dn):
    RB = 1000

    def body(su_ref, sd_ref, p_ref, iu_ref, id_ref, o_ref, ob_ref):
        iu = iu_ref[...]
        idn = id_ref[...]
        sq = None
        xs = []
        for k in range(4):
            xu = su_ref[0, k] * iu
            xd = sd_ref[0, k] * idn
            xb = p_ref[k]
            xs.append((xu, xd, xb))
            t = jnp.sum(xu * xu + xd * xd + xb * xb, axis=-1, keepdims=True)
            sq = t if sq is None else sq + t
        scale = 1.0 / jnp.maximum(jnp.sqrt(sq), 1e-12)
        for k, (xu, xd, xb) in enumerate(xs):
            for off, v in ((0, xu), (512, xd), (1024, xb)):
                vv = v * scale
                vv = jnp.where(vv >= 0, vv, 0.1 * vv)
                o_ref[:, off + k * 128: off + (k + 1) * 128] = vv
                ob_ref[:, off + k * 128: off + (k + 1) * 128] = (
                    vv.astype(jnp.bfloat16))

    vspec = pl.BlockSpec((RB, 1), lambda i: (i, 0))
    return pl.pallas_call(
        body,
        grid=(_N // RB,),
        in_specs=[
            pl.BlockSpec((1, 4, RB, 128), lambda i: (0, 0, i, 0)),
            pl.BlockSpec((1, 4, RB, 128), lambda i: (1, 0, i, 0)),
            pl.BlockSpec((4, RB, 128), lambda i: (2, i, 0)),
            vspec, vspec,
        ],
        out_specs=[
            pl.BlockSpec((RB, 1536), lambda i: (i, 0)),
            pl.BlockSpec((RB, 1536), lambda i: (i, 0)),
        ],
        out_shape=[
            jax.ShapeDtypeStruct((_N, 1536), jnp.float32),
            jax.ShapeDtypeStruct((_N, 1536), jnp.bfloat16),
        ],
    )(s, s, proj12, inv_up, inv_dn)


def _mean_w(ug1, ug2, ug3):
    g = jnp.stack([ug1.reshape(300, 400), ug2.reshape(300, 400),
                   ug3.reshape(300, 400)], axis=0)

    def body(g_ref, o_ref):
        o_ref[...] = (g_ref[0] + g_ref[1] + g_ref[2]) * (1.0 / 3.0)

    mg = pl.pallas_call(
        body,
        out_shape=jax.ShapeDtypeStruct((300, 400), jnp.float32),
    )(g)
    return jnp.concatenate([mg.reshape(_NUM_GENE),
                            jnp.ones((_NUM_DRUG,), jnp.float32)], axis=0)


def _to64(p8):
    return (p8.reshape(2, 4, _N, 2, _CW)
            .transpose(0, 1, 3, 2, 4)
            .reshape(2, _NCH, _N, _CW))


def kernel(x, edge_index, Wu1, Wd1, Wb1, ug1, dg1, Wu2, Wd2, Wb2, ug2, dg2,
           Wu3, Wd3, Wb3, ug3, dg3):
    ei_r = edge_index.astype(jnp.int32).reshape(2, _NS, _NBATCH, _BB)
    drug_ones = jnp.ones((_NUM_DRUG,), jnp.float32)

    hb = x.astype(jnp.bfloat16)
    h = None
    inv_up = inv_dn = None
    deg = None
    for (Wu, Wd, Wb, ug, dg) in ((Wu1, Wd1, Wb1, ug1, dg1),
                                 (Wu2, Wd2, Wb2, ug2, dg2),
                                 (Wu3, Wd3, Wb3, ug3, dg3)):
        W = jnp.concatenate([Wu.T, Wd.T, Wb.T], axis=1)
        proj12 = _matmul12(hb, W.astype(jnp.bfloat16))
        tab = _to64(proj12[:8])
        up_w = jnp.concatenate([ug, drug_ones], axis=0)
        dn_w = jnp.concatenate([dg, drug_ones], axis=0)
        w_r = jnp.stack([up_w, dn_w], axis=0).reshape(2, _NS, _NBATCH, _BB)
        if inv_up is None:
            s, deg = _sc_aggregate(tab, ei_r, w_r, True)
        else:
            s = _sc_aggregate(tab, ei_r, w_r, False)
        if deg is not None and inv_up is None:
            deg_up = deg[0, :_N, 0:1]
            deg_dn = deg[1, :_N, 0:1]
            inv_up = jnp.where(deg_up > 0, 1.0 / deg_up, 0.0)
            inv_dn = jnp.where(deg_dn > 0, 1.0 / deg_dn, 0.0)
        h, hb = _epilogue_c(s, proj12, inv_up, inv_dn)

    mean_up = _mean_w(ug1, ug2, ug3)
    mean_down = _mean_w(dg1, dg2, dg3)
    return h, mean_up, mean_down


# speedup vs baseline: 1.2136x; 1.2136x over previous
import functools

import jax
import jax.numpy as jnp
from jax import lax
from jax.experimental import pallas as pl
from jax.experimental.pallas import tpu as pltpu
from jax.experimental.pallas import tpu_sc as plsc

_N = 10000
_E = 160000
_NUM_GENE = 120000
_NUM_DRUG = _E - _NUM_GENE

_NC, _NS, _L = 2, 16, 16
_EPT = _E // _NS
_BB = 80
_NBATCH = _EPT // _BB
_NPAD = 10240
_RPT = _NPAD // _NS
_ZR = 128
_CW = 64
_NCH = 512 // _CW


def _matmul12(h, W):
    N, K = h.shape
    RB = 1000

    def body(h_ref, w_ref, o_ref):
        o_ref[0] = jnp.dot(h_ref[...], w_ref[...],
                           preferred_element_type=jnp.float32)

    return pl.pallas_call(
        body,
        grid=(N // RB, 12),
        in_specs=[
            pl.BlockSpec((RB, K), lambda i, j: (i, 0)),
            pl.BlockSpec((K, 128), lambda i, j: (0, j)),
        ],
        out_specs=pl.BlockSpec((1, RB, 128), lambda i, j: (j, i, 0)),
        out_shape=jax.ShapeDtypeStruct((12, N, 128), jnp.float32),
    )(h, W)


def _sc_aggregate(tab, ei_r, w_r, do_deg):
    mesh = plsc.VectorSubcoreMesh(core_axis_name="c", subcore_axis_name="s")

    @functools.partial(
        pl.kernel,
        out_type=([jax.ShapeDtypeStruct((2, 4, _NPAD, 128), jnp.float32),
                   jax.ShapeDtypeStruct((2, _NPAD, _CW), jnp.float32)]
                  if do_deg else
                  jax.ShapeDtypeStruct((2, 4, _NPAD, 128), jnp.float32)),
        mesh=mesh,
        compiler_params=pltpu.CompilerParams(use_tc_tiling_on_sc=False),
        scratch_types=[
            pltpu.VMEM((_NBATCH, _BB), jnp.int32),
            pltpu.VMEM((_NBATCH, _BB), jnp.int32),
            pltpu.VMEM((_NBATCH, _BB), jnp.float32),
            pltpu.VMEM((4, _BB, _CW), jnp.float32),
            pltpu.VMEM((_ZR, _CW), jnp.float32),
            pltpu.VMEM((_ZR, _CW), jnp.float32),
            pltpu.VMEM((1, _BB), jnp.int32),
            pltpu.VMEM_SHARED((_NPAD, _CW), jnp.float32),
            [pltpu.SemaphoreType.DMA] * 4,
            [pltpu.SemaphoreType.DMA] * 4,
        ],
    )
    def agg(tab_h, ei_h, w_h, out_h, *rest):
        if do_deg:
            (deg_h, gidx_v, sidx_v, w_v, rows_v,
             z_v, cp_v, pad_v, acc_sh, gsems, ssems) = rest
        else:
            (gidx_v, sidx_v, w_v, rows_v,
             z_v, cp_v, pad_v, acc_sh, gsems, ssems) = rest
        c = lax.axis_index("c")
        s = lax.axis_index("s")
        pltpu.sync_copy(ei_h.at[c, s], gidx_v)
        pltpu.sync_copy(ei_h.at[1 - c, s], sidx_v)
        pltpu.sync_copy(w_h.at[c, s], w_v)
        zv = jnp.zeros((_L,), jnp.float32)
        for r in range(_ZR):
            for j in range(_CW // _L):
                z_v[r, pl.ds(j * _L, _L)] = zv
        padv = jnp.full((_L,), _NPAD - 8, jnp.int32)
        for j in range(_BB // _L):
            pad_v[0, pl.ds(j * _L, _L)] = padv

        def chunk_body(ch, carry):
            for r in range(_RPT // _ZR):
                pltpu.sync_copy(
                    z_v, acc_sh.at[pl.ds(s * _RPT + r * _ZR, _ZR)])
            plsc.subcore_barrier()

            def gather(bi, slot):
                pltpu.async_copy(
                    tab_h.at[c, ch].at[gidx_v.at[bi]], rows_v.at[slot],
                    gsems[slot])

            def wait_gather(slot):
                pltpu.make_async_copy(
                    tab_h.at[c, ch].at[gidx_v.at[0]], rows_v.at[slot],
                    gsems[slot]).wait()

            def scatter(bi, slot):
                pltpu.async_copy(rows_v.at[slot], acc_sh.at[sidx_v.at[bi]],
                                 ssems[slot], add=True)

            def wait_scatter(slot):
                pltpu.make_async_copy(rows_v.at[slot],
                                      acc_sh.at[sidx_v.at[0]],
                                      ssems[slot]).wait()

            def scale(bi, slot):
                def scale16(g, carry2):
                    w16 = w_v[bi, pl.ds(g * 16, 16)]
                    for b in range(16):
                        e = g * 16 + b
                        w_s = w16[b]
                        for j in range(_CW // _L):
                            sl = pl.ds(j * _L, _L)
                            rows_v[slot, e, sl] = rows_v[slot, e, sl] * w_s
                    return carry2
                lax.fori_loop(0, _BB // 16, scale16, 0)

            for k in (1, 2, 3):
                pltpu.async_copy(rows_v.at[k], acc_sh.at[pad_v.at[0]],
                                 ssems[k], add=True)
            for k in (0, 1, 2):
                gather(k, k)

            def group_body(p, carry2):
                for q in range(4):
                    bi = p * 4 + q
                    wait_gather(q)
                    scale(bi, q)
                    scatter(bi, q)
                    pk = (q + 3) % 4
                    wait_scatter(pk)

                    @pl.when(bi + 3 < _NBATCH)
                    def _():
                        gather(bi + 3, pk)
                return carry2

            lax.fori_loop(0, (_NBATCH - 1) // 4, group_body, 0)
            wait_gather(0)
            scale(_NBATCH - 1, 0)
            scatter(_NBATCH - 1, 0)
            for k in range(4):
                wait_scatter(k)

            plsc.subcore_barrier()
            ch128 = lax.div(ch, 2)
            hf = lax.rem(ch, 2)
            for r in range(_RPT // _ZR):
                row0 = s * _RPT + r * _ZR
                pltpu.sync_copy(acc_sh.at[pl.ds(row0, _ZR)], cp_v)
                pltpu.sync_copy(
                    cp_v,
                    out_h.at[c, ch128, pl.ds(row0, _ZR),
                             pl.ds(hf * _CW, _CW)])
            return carry

        lax.fori_loop(0, _NCH, chunk_body, 0)

        if not do_deg:
            return
        one = jnp.full((_L,), 1.0, jnp.float32)
        for j in range(_CW // _L):
            z_v[0, pl.ds(j * _L, _L)] = one

        def ones_fill(r, carry):
            for j in range(_CW // _L):
                rows_v[0, r, pl.ds(j * _L, _L)] = z_v[0, pl.ds(j * _L, _L)]
            return carry
        lax.fori_loop(0, _BB, ones_fill, 0)
        zv2 = jnp.zeros((_L,), jnp.float32)
        for j in range(_CW // _L):
            z_v[0, pl.ds(j * _L, _L)] = zv2
        for r in range(_RPT // _ZR):
            pltpu.sync_copy(z_v, acc_sh.at[pl.ds(s * _RPT + r * _ZR, _ZR)])
        plsc.subcore_barrier()

        def dscatter(bi, k):
            pltpu.async_copy(rows_v.at[0], acc_sh.at[sidx_v.at[bi]],
                             ssems[k], add=True)

        def dwait(k):
            pltpu.make_async_copy(rows_v.at[0], acc_sh.at[sidx_v.at[0]],
                                  ssems[k]).wait()

        def deg_group(p, carry):
            for q in range(4):
                dscatter(p * 4 + q, q)
            for q in range(4):
                dwait(q)
            return carry
        lax.fori_loop(0, (_NBATCH - 1) // 4, deg_group, 0)
        dscatter(_NBATCH - 1, 0)
        dwait(0)
        plsc.subcore_barrier()
        for r in range(_RPT // _ZR):
            row0 = s * _RPT + r * _ZR
            pltpu.sync_copy(acc_sh.at[pl.ds(row0, _ZR)], cp_v)
            pltpu.sync_copy(cp_v, deg_h.at[c].at[pl.ds(row0, _ZR)])

    return agg(tab, ei_r, w_r)


def _epilogue_c(s, proj12, inv_up, inv_dn):
    RB = 1000

    def body(su_ref, sd_ref, p_ref, iu_ref, id_ref, o_ref, ob_ref):
        iu = iu_ref[...]
        idn = id_ref[...]
        sq = None
        xs = []
        for k in range(4):
            xu = su_ref[0, k] * iu
            xd = sd_ref[0, k] * idn
            xb = p_ref[k]
            xs.append((xu, xd, xb))
            t = jnp.sum(xu * xu + xd * xd + xb * xb, axis=-1, keepdims=True)
            sq = t if sq is None else sq + t
        scale = 1.0 / jnp.maximum(jnp.sqrt(sq), 1e-12)
        for k, (xu, xd, xb) in enumerate(xs):
            for off, v in ((0, xu), (512, xd), (1024, xb)):
                vv = v * scale
                vv = jnp.where(vv >= 0, vv, 0.1 * vv)
                o_ref[:, off + k * 128: off + (k + 1) * 128] = vv
                ob_ref[:, off + k * 128: off + (k + 1) * 128] = (
                    vv.astype(jnp.bfloat16))

    vspec = pl.BlockSpec((RB, 1), lambda i: (i, 0))
    return pl.pallas_call(
        body,
        grid=(_N // RB,),
        in_specs=[
            pl.BlockSpec((1, 4, RB, 128), lambda i: (0, 0, i, 0)),
            pl.BlockSpec((1, 4, RB, 128), lambda i: (1, 0, i, 0)),
            pl.BlockSpec((4, RB, 128), lambda i: (2, i, 0)),
            vspec, vspec,
        ],
        out_specs=[
            pl.BlockSpec((RB, 1536), lambda i: (i, 0)),
            pl.BlockSpec((RB, 1536), lambda i: (i, 0)),
        ],
        out_shape=[
            jax.ShapeDtypeStruct((_N, 1536), jnp.float32),
            jax.ShapeDtypeStruct((_N, 1536), jnp.bfloat16),
        ],
    )(s, s, proj12, inv_up, inv_dn)


def _mean_w(ug1, ug2, ug3):
    g = jnp.stack([ug1.reshape(300, 400), ug2.reshape(300, 400),
                   ug3.reshape(300, 400)], axis=0)

    def body(g_ref, o_ref):
        o_ref[...] = (g_ref[0] + g_ref[1] + g_ref[2]) * (1.0 / 3.0)

    mg = pl.pallas_call(
        body,
        out_shape=jax.ShapeDtypeStruct((300, 400), jnp.float32),
    )(g)
    return jnp.concatenate([mg.reshape(_NUM_GENE),
                            jnp.ones((_NUM_DRUG,), jnp.float32)], axis=0)


def _to64(p8):
    return (p8.reshape(2, 4, _N, 2, _CW)
            .transpose(0, 1, 3, 2, 4)
            .reshape(2, _NCH, _N, _CW))


def kernel(x, edge_index, Wu1, Wd1, Wb1, ug1, dg1, Wu2, Wd2, Wb2, ug2, dg2,
           Wu3, Wd3, Wb3, ug3, dg3):
    src = edge_index[0]
    dst = edge_index[1]
    ones_e = jnp.ones((_E,), jnp.float32)
    deg_up = jax.ops.segment_sum(ones_e, dst, num_segments=_N)
    deg_dn = jax.ops.segment_sum(ones_e, src, num_segments=_N)
    inv_up = jnp.where(deg_up > 0, 1.0 / deg_up, 0.0).reshape(_N, 1)
    inv_dn = jnp.where(deg_dn > 0, 1.0 / deg_dn, 0.0).reshape(_N, 1)

    ei_r = edge_index.astype(jnp.int32).reshape(2, _NS, _NBATCH, _BB)
    drug_ones = jnp.ones((_NUM_DRUG,), jnp.float32)

    hb = x.astype(jnp.bfloat16)
    h = None
    for (Wu, Wd, Wb, ug, dg) in ((Wu1, Wd1, Wb1, ug1, dg1),
                                 (Wu2, Wd2, Wb2, ug2, dg2),
                                 (Wu3, Wd3, Wb3, ug3, dg3)):
        W = jnp.concatenate([Wu.T, Wd.T, Wb.T], axis=1)
        proj12 = _matmul12(hb, W.astype(jnp.bfloat16))
        tab = _to64(proj12[:8])
        up_w = jnp.concatenate([ug, drug_ones], axis=0)
        dn_w = jnp.concatenate([dg, drug_ones], axis=0)
        w_r = jnp.stack([up_w, dn_w], axis=0).reshape(2, _NS, _NBATCH, _BB)
        s = _sc_aggregate(tab, ei_r, w_r, False)
        h, hb = _epilogue_c(s, proj12, inv_up, inv_dn)

    mean_up = _mean_w(ug1, ug2, ug3)
    mean_down = _mean_w(dg1, dg2, dg3)
    return h, mean_up, mean_down


# speedup vs baseline: 1.2138x; 1.0002x over previous
import functools

import jax
import jax.numpy as jnp
from jax import lax
from jax.experimental import pallas as pl
from jax.experimental.pallas import tpu as pltpu
from jax.experimental.pallas import tpu_sc as plsc

_N = 10000
_E = 160000
_NUM_GENE = 120000
_NUM_DRUG = _E - _NUM_GENE

_NC, _NS, _L = 2, 16, 16
_EPT = _E // _NS
_BB = 80
_NBATCH = _EPT // _BB
_NPAD = 10240
_RPT = _NPAD // _NS
_ZR = 128
_CW = 64
_NCH = 512 // _CW


def _matmul12(h, W):
    N, K = h.shape
    RB = 1000

    def body(h_ref, w_ref, o_ref):
        o_ref[0] = jnp.dot(h_ref[...], w_ref[...],
                           preferred_element_type=jnp.float32)

    return pl.pallas_call(
        body,
        grid=(N // RB, 12),
        in_specs=[
            pl.BlockSpec((RB, K), lambda i, j: (i, 0)),
            pl.BlockSpec((K, 128), lambda i, j: (0, j)),
        ],
        out_specs=pl.BlockSpec((1, RB, 128), lambda i, j: (j, i, 0)),
        out_shape=jax.ShapeDtypeStruct((12, N, 128), jnp.float32),
    )(h, W)


def _sc_aggregate(tab, ei_r, w_r):
    mesh = plsc.VectorSubcoreMesh(core_axis_name="c", subcore_axis_name="s")

    @functools.partial(
        pl.kernel,
        out_type=jax.ShapeDtypeStruct((2, 4, _NPAD, 128), jnp.float32),
        mesh=mesh,
        compiler_params=pltpu.CompilerParams(use_tc_tiling_on_sc=False),
        scratch_types=[
            pltpu.VMEM((_NBATCH, _BB), jnp.int32),
            pltpu.VMEM((_NBATCH, _BB), jnp.int32),
            pltpu.VMEM((_NBATCH, _BB), jnp.float32),
            pltpu.VMEM((4, _BB, _CW), jnp.float32),
            pltpu.VMEM((_ZR, _CW), jnp.float32),
            pltpu.VMEM((_ZR, _CW), jnp.float32),
            pltpu.VMEM((1, _BB), jnp.int32),
            pltpu.VMEM_SHARED((_NPAD, _CW), jnp.float32),
            [pltpu.SemaphoreType.DMA] * 4,
            [pltpu.SemaphoreType.DMA] * 4,
        ],
    )
    def agg(tab_h, ei_h, w_h, out_h, gidx_v, sidx_v, w_v, rows_v,
            z_v, cp_v, pad_v, acc_sh, gsems, ssems):
        c = lax.axis_index("c")
        s = lax.axis_index("s")
        pltpu.sync_copy(ei_h.at[c, s], gidx_v)
        pltpu.sync_copy(ei_h.at[1 - c, s], sidx_v)
        pltpu.sync_copy(w_h.at[c, s], w_v)
        zv = jnp.zeros((_L,), jnp.float32)
        for r in range(_ZR):
            for j in range(_CW // _L):
                z_v[r, pl.ds(j * _L, _L)] = zv
        padv = jnp.full((_L,), _NPAD - 8, jnp.int32)
        for j in range(_BB // _L):
            pad_v[0, pl.ds(j * _L, _L)] = padv

        def chunk_body(ch, carry):
            for r in range(_RPT // _ZR):
                pltpu.sync_copy(
                    z_v, acc_sh.at[pl.ds(s * _RPT + r * _ZR, _ZR)])
            plsc.subcore_barrier()

            def gather(bi, slot):
                pltpu.async_copy(
                    tab_h.at[c, ch].at[gidx_v.at[bi]], rows_v.at[slot],
                    gsems[slot])

            def wait_gather(slot):
                pltpu.make_async_copy(
                    tab_h.at[c, ch].at[gidx_v.at[0]], rows_v.at[slot],
                    gsems[slot]).wait()

            def scatter(bi, slot):
                pltpu.async_copy(rows_v.at[slot], acc_sh.at[sidx_v.at[bi]],
                                 ssems[slot], add=True)

            def wait_scatter(slot):
                pltpu.make_async_copy(rows_v.at[slot],
                                      acc_sh.at[sidx_v.at[0]],
                                      ssems[slot]).wait()

            def scale(bi, slot):
                def scale16(g, carry2):
                    w16 = w_v[bi, pl.ds(g * 16, 16)]
                    for b in range(16):
                        e = g * 16 + b
                        w_s = w16[b]
                        for j in range(_CW // _L):
                            sl = pl.ds(j * _L, _L)
                            rows_v[slot, e, sl] = rows_v[slot, e, sl] * w_s
                    return carry2
                lax.fori_loop(0, _BB // 16, scale16, 0)

            for k in (1, 2, 3):
                pltpu.async_copy(rows_v.at[k], acc_sh.at[pad_v.at[0]],
                                 ssems[k], add=True)
            for k in (0, 1, 2):
                gather(k, k)

            def group_body(p, carry2):
                for q in range(4):
                    bi = p * 4 + q
                    wait_gather(q)
                    scale(bi, q)
                    scatter(bi, q)
                    pk = (q + 3) % 4
                    wait_scatter(pk)

                    @pl.when(bi + 3 < _NBATCH)
                    def _():
                        gather(bi + 3, pk)
                return carry2

            lax.fori_loop(0, (_NBATCH - 1) // 4, group_body, 0)
            wait_gather(0)
            scale(_NBATCH - 1, 0)
            scatter(_NBATCH - 1, 0)
            for k in range(4):
                wait_scatter(k)

            plsc.subcore_barrier()
            ch128 = lax.div(ch, 2)
            hf = lax.rem(ch, 2)
            for r in range(_RPT // _ZR):
                row0 = s * _RPT + r * _ZR
                pltpu.sync_copy(acc_sh.at[pl.ds(row0, _ZR)], cp_v)
                pltpu.sync_copy(
                    cp_v,
                    out_h.at[c, ch128, pl.ds(row0, _ZR),
                             pl.ds(hf * _CW, _CW)])
            return carry

        lax.fori_loop(0, _NCH, chunk_body, 0)

    return agg(tab, ei_r, w_r)


def _epilogue_c(s, proj12, inv_up, inv_dn):
    RB = 1000

    def body(su_ref, sd_ref, p_ref, iu_ref, id_ref, o_ref, ob_ref):
        iu = iu_ref[...]
        idn = id_ref[...]
        sq = None
        xs = []
        for k in range(4):
            xu = su_ref[0, k] * iu
            xd = sd_ref[0, k] * idn
            xb = p_ref[k]
            xs.append((xu, xd, xb))
            t = jnp.sum(xu * xu + xd * xd + xb * xb, axis=-1, keepdims=True)
            sq = t if sq is None else sq + t
        scale = 1.0 / jnp.maximum(jnp.sqrt(sq), 1e-12)
        for k, (xu, xd, xb) in enumerate(xs):
            for off, v in ((0, xu), (512, xd), (1024, xb)):
                vv = v * scale
                vv = jnp.where(vv >= 0, vv, 0.1 * vv)
                o_ref[:, off + k * 128: off + (k + 1) * 128] = vv
                ob_ref[:, off + k * 128: off + (k + 1) * 128] = (
                    vv.astype(jnp.bfloat16))

    vspec = pl.BlockSpec((RB, 1), lambda i: (i, 0))
    return pl.pallas_call(
        body,
        grid=(_N // RB,),
        in_specs=[
            pl.BlockSpec((1, 4, RB, 128), lambda i: (0, 0, i, 0)),
            pl.BlockSpec((1, 4, RB, 128), lambda i: (1, 0, i, 0)),
            pl.BlockSpec((4, RB, 128), lambda i: (2, i, 0)),
            vspec, vspec,
        ],
        out_specs=[
            pl.BlockSpec((RB, 1536), lambda i: (i, 0)),
            pl.BlockSpec((RB, 1536), lambda i: (i, 0)),
        ],
        out_shape=[
            jax.ShapeDtypeStruct((_N, 1536), jnp.float32),
            jax.ShapeDtypeStruct((_N, 1536), jnp.bfloat16),
        ],
    )(s, s, proj12, inv_up, inv_dn)


def _mean_w(ug1, ug2, ug3):
    g = jnp.stack([ug1.reshape(300, 400), ug2.reshape(300, 400),
                   ug3.reshape(300, 400)], axis=0)

    def body(g_ref, o_ref):
        o_ref[...] = (g_ref[0] + g_ref[1] + g_ref[2]) * (1.0 / 3.0)

    mg = pl.pallas_call(
        body,
        out_shape=jax.ShapeDtypeStruct((300, 400), jnp.float32),
    )(g)
    return jnp.concatenate([mg.reshape(_NUM_GENE),
                            jnp.ones((_NUM_DRUG,), jnp.float32)], axis=0)


def _to64(p8):
    return (p8.reshape(2, 4, _N, 2, _CW)
            .transpose(0, 1, 3, 2, 4)
            .reshape(2, _NCH, _N, _CW))


def kernel(x, edge_index, Wu1, Wd1, Wb1, ug1, dg1, Wu2, Wd2, Wb2, ug2, dg2,
           Wu3, Wd3, Wb3, ug3, dg3):
    src = edge_index[0]
    dst = edge_index[1]
    ones_e = jnp.ones((_E,), jnp.float32)
    deg_up = jax.ops.segment_sum(ones_e, dst, num_segments=_N)
    deg_dn = jax.ops.segment_sum(ones_e, src, num_segments=_N)
    inv_up = jnp.where(deg_up > 0, 1.0 / deg_up, 0.0).reshape(_N, 1)
    inv_dn = jnp.where(deg_dn > 0, 1.0 / deg_dn, 0.0).reshape(_N, 1)

    ei_r = edge_index.astype(jnp.int32).reshape(2, _NS, _NBATCH, _BB)
    drug_ones = jnp.ones((_NUM_DRUG,), jnp.float32)

    hb = x.astype(jnp.bfloat16)
    h = None
    for (Wu, Wd, Wb, ug, dg) in ((Wu1, Wd1, Wb1, ug1, dg1),
                                 (Wu2, Wd2, Wb2, ug2, dg2),
                                 (Wu3, Wd3, Wb3, ug3, dg3)):
        W = jnp.concatenate([Wu.T, Wd.T, Wb.T], axis=1)
        proj12 = _matmul12(hb, W.astype(jnp.bfloat16))
        tab = _to64(proj12[:8])
        up_w = jnp.concatenate([ug, drug_ones], axis=0)
        dn_w = jnp.concatenate([dg, drug_ones], axis=0)
        w_r = jnp.stack([up_w, dn_w], axis=0).reshape(2, _NS, _NBATCH, _BB)
        s = _sc_aggregate(tab, ei_r, w_r)
        h, hb = _epilogue_c(s, proj12, inv_up, inv_dn)

    mean_up = _mean_w(ug1, ug2, ug3)
    mean_down = _mean_w(dg1, dg2, dg3)
    return h, mean_up, mean_down
